# baseline (speedup 1.0000x reference)
import functools

import jax
import jax.numpy as jnp
from jax import lax
from jax.experimental import pallas as pl
from jax.experimental.pallas import tpu as pltpu
from jax.experimental.pallas import tpu_sc as plsc

DIM = 4096
N_EXPERTS = 64
TOPK = 8
BT = 1024
NW = 32
CH = 512
LANES = 16


def _score_kernel(x_ref, w_ref, p_ref):
    x = x_ref[...]
    w = w_ref[...]
    scores = jax.lax.dot_general(
        w, x, (((1,), (1,)), ((), ())), preferred_element_type=jnp.float32
    )
    m = jnp.max(scores, axis=0, keepdims=True)
    e = jnp.exp(scores - m)
    probs = e / jnp.sum(e, axis=0, keepdims=True)
    p_ref[0] = probs[:, :CH]
    p_ref[1] = probs[:, CH:]


def _tc_probs(x, weight):
    n_tokens = x.shape[0]
    return pl.pallas_call(
        _score_kernel,
        grid=(n_tokens // BT,),
        in_specs=[
            pl.BlockSpec((BT, DIM), lambda i: (i, 0)),
            pl.BlockSpec((N_EXPERTS, DIM), lambda i: (0, 0)),
        ],
        out_specs=pl.BlockSpec((BT // CH, N_EXPERTS, CH), lambda i: (i, 0, 0)),
        out_shape=jax.ShapeDtypeStruct(
            (n_tokens // CH, N_EXPERTS, CH), jnp.float32
        ),
    )(x, weight)


def _sc_topk_build():
    mesh = plsc.VectorSubcoreMesh(core_axis_name="c", subcore_axis_name="s")

    @functools.partial(
        pl.kernel,
        mesh=mesh,
        out_type=[
            jax.ShapeDtypeStruct((NW, TOPK, CH), jnp.float32),
            jax.ShapeDtypeStruct((NW, TOPK, CH), jnp.int32),
        ],
        scratch_types=[
            pltpu.VMEM((N_EXPERTS, CH), jnp.float32),
            pltpu.VMEM((TOPK, CH), jnp.float32),
            pltpu.VMEM((TOPK, CH), jnp.int32),
        ],
    )
    def sc_topk(p_hbm, wout_hbm, iout_hbm, p_v, w_v, i_v):
        wid = lax.axis_index("s") * 2 + lax.axis_index("c")
        pltpu.sync_copy(p_hbm.at[wid], p_v)

        def group_body(g, _):
            sl = pl.ds(pl.multiple_of(g * LANES, LANES), LANES)
            init = (
                tuple(jnp.full((LANES,), -1.0, jnp.float32) for _ in range(TOPK)),
                tuple(jnp.zeros((LANES,), jnp.int32) for _ in range(TOPK)),
            )

            def expert_body(i, carry):
                vals, idxs = carry
                e = N_EXPERTS - 1 - i
                v = p_v[e, sl]
                ei = jnp.full((LANES,), e, jnp.int32)
                nv_l, ni_l = [], []
                for j in range(TOPK):
                    swap = v >= vals[j]
                    nv = jnp.where(swap, v, vals[j])
                    pv = jnp.where(swap, vals[j], v)
                    ni = jnp.where(swap, ei, idxs[j])
                    pi = jnp.where(swap, idxs[j], ei)
                    nv_l.append(nv)
                    ni_l.append(ni)
                    v, ei = pv, pi
                return tuple(nv_l), tuple(ni_l)

            vals, idxs = lax.fori_loop(0, N_EXPERTS, expert_body, init)
            for j in range(TOPK):
                w_v[j, sl] = vals[j]
                i_v[j, sl] = idxs[j]
            return 0

        lax.fori_loop(0, CH // LANES, group_body, 0)
        pltpu.sync_copy(w_v, wout_hbm.at[wid])
        pltpu.sync_copy(i_v, iout_hbm.at[wid])

    return sc_topk


_sc_topk = _sc_topk_build()


def kernel(x, weight):
    n_tokens = x.shape[0]
    probs = _tc_probs(x, weight)
    wout, iout = _sc_topk(probs)
    w = wout.transpose(0, 2, 1).reshape(n_tokens, TOPK)
    i = iout.transpose(0, 2, 1).reshape(n_tokens, TOPK)
    return w, i


# speedup vs baseline: 1.0057x; 1.0057x over previous
import functools

import jax
import jax.numpy as jnp
from jax import lax
from jax.experimental import pallas as pl
from jax.experimental.pallas import tpu as pltpu
from jax.experimental.pallas import tpu_sc as plsc

DIM = 4096
N_EXPERTS = 64
TOPK = 8
BT = 1024
NW = 32
CH = 512
LANES = 16


def _score_kernel(x_ref, w_ref, p_ref):
    x = x_ref[...]
    w = w_ref[...]
    scores = jax.lax.dot_general(
        w, x, (((1,), (1,)), ((), ())), preferred_element_type=jnp.float32
    )
    m = jnp.max(scores, axis=0, keepdims=True)
    e = jnp.exp(scores - m)
    probs = e / jnp.sum(e, axis=0, keepdims=True)
    p_ref[0] = probs[:, :CH]
    p_ref[1] = probs[:, CH:]


def _tc_probs(x, weight):
    n_tokens = x.shape[0]
    return pl.pallas_call(
        _score_kernel,
        grid=(n_tokens // BT,),
        in_specs=[
            pl.BlockSpec((BT, DIM), lambda i: (i, 0)),
            pl.BlockSpec((N_EXPERTS, DIM), lambda i: (0, 0)),
        ],
        out_specs=pl.BlockSpec((BT // CH, N_EXPERTS, CH), lambda i: (i, 0, 0)),
        out_shape=jax.ShapeDtypeStruct(
            (n_tokens // CH, N_EXPERTS, CH), jnp.float32
        ),
    )(x, weight)


def _sc_topk_build():
    mesh = plsc.VectorSubcoreMesh(core_axis_name="c", subcore_axis_name="s")

    @functools.partial(
        pl.kernel,
        mesh=mesh,
        out_type=[
            jax.ShapeDtypeStruct((NW, TOPK, CH), jnp.float32),
            jax.ShapeDtypeStruct((NW, TOPK, CH), jnp.int32),
        ],
        scratch_types=[
            pltpu.VMEM((N_EXPERTS, CH), jnp.float32),
            pltpu.VMEM((TOPK, CH), jnp.float32),
            pltpu.VMEM((TOPK, CH), jnp.int32),
        ],
    )
    def sc_topk(p_hbm, wout_hbm, iout_hbm, p_v, w_v, i_v):
        wid = lax.axis_index("s") * 2 + lax.axis_index("c")
        pltpu.sync_copy(p_hbm.at[wid], p_v)

        NG = CH // LANES
        NI = 2

        def group_body(g, _):
            sls = [
                pl.ds(pl.multiple_of((g * NI + k) * LANES, LANES), LANES)
                for k in range(NI)
            ]
            init = (
                tuple(jnp.full((LANES,), -1.0, jnp.float32)
                      for _ in range(NI * TOPK)),
                tuple(jnp.zeros((LANES,), jnp.int32)
                      for _ in range(NI * TOPK)),
            )

            def expert_body(i, carry):
                vals, idxs = carry
                e = N_EXPERTS - 1 - i
                ei = jnp.full((LANES,), e, jnp.int32)
                nv_l, ni_l = [], []
                for k in range(NI):
                    v = p_v[e, sls[k]]
                    eik = ei
                    for j in range(TOPK):
                        jj = k * TOPK + j
                        swap = v >= vals[jj]
                        nv = jnp.where(swap, v, vals[jj])
                        pv = jnp.where(swap, vals[jj], v)
                        ni = jnp.where(swap, eik, idxs[jj])
                        pi = jnp.where(swap, idxs[jj], eik)
                        nv_l.append(nv)
                        ni_l.append(ni)
                        v, eik = pv, pi
                return tuple(nv_l), tuple(ni_l)

            vals, idxs = lax.fori_loop(0, N_EXPERTS, expert_body, init)
            for k in range(NI):
                for j in range(TOPK):
                    w_v[j, sls[k]] = vals[k * TOPK + j]
                    i_v[j, sls[k]] = idxs[k * TOPK + j]
            return 0

        lax.fori_loop(0, NG // NI, group_body, 0)
        pltpu.sync_copy(w_v, wout_hbm.at[wid])
        pltpu.sync_copy(i_v, iout_hbm.at[wid])

    return sc_topk


_sc_topk = _sc_topk_build()


def kernel(x, weight):
    n_tokens = x.shape[0]
    probs = _tc_probs(x, weight)
    wout, iout = _sc_topk(probs)
    w = wout.transpose(0, 2, 1).reshape(n_tokens, TOPK)
    i = iout.transpose(0, 2, 1).reshape(n_tokens, TOPK)
    return w, i
